# Initial kernel scaffold; baseline (speedup 1.0000x reference)
import jax
import jax.numpy as jnp
from jax.experimental import pallas as pl


def kernel(x, edge_index, Wl1, bl1, Wr1, Wl2, bl2, Wr2):
    raise NotImplementedError("write your pallas kernel here")


# speedup vs baseline: 3.1067x; 3.1067x over previous
import functools

import jax
import jax.numpy as jnp
from jax import lax
from jax.experimental import pallas as pl
from jax.experimental.pallas import tpu as pltpu
from jax.experimental.pallas import tpu_sc as plsc

N = 10000
E = 320000
D = 128

NC = 2
NS = 16
NW = NC * NS
CHUNK = 128
GRP = 8
CHUNKS = 80
GRPS = CHUNKS // GRP
EPAD = NW * CHUNKS * CHUNK
NPT = 632
NPAD = NS * NPT


def _segsum_body(with_hist, *refs):
    if with_hist:
        (table, srcg, dstg, zeros, zeros1,
         out, degout, acc, idx_s, idx_d, buf, hist) = refs
    else:
        (table, srcg, dstg, zeros,
         out, acc, idx_s, idx_d, buf) = refs

    c = lax.axis_index("c")
    s = lax.axis_index("s")
    w = c * NS + s

    rows = pl.ds(s * NPT, NPT)
    pltpu.sync_copy(zeros.at[rows], acc.at[rows])
    if with_hist:
        pltpu.sync_copy(zeros1, hist)
    plsc.subcore_barrier()

    ones = jnp.full((16,), 1.0, jnp.float32)

    def body(g, carry):
        pltpu.sync_copy(srcg.at[w, pl.ds(g * GRP, GRP)], idx_s)
        pltpu.sync_copy(dstg.at[w, pl.ds(g * GRP, GRP)], idx_d)

        def inner(j, carry2):
            pltpu.sync_copy(table.at[idx_s.at[j]], buf)
            pltpu.sync_copy(buf, acc.at[idx_d.at[j]], add=True)
            if with_hist:
                for k in range(CHUNK // 16):
                    v = idx_d[j, pl.ds(k * 16, 16)]
                    plsc.addupdate_scatter(hist, [v], ones)
            return carry2

        lax.fori_loop(0, GRP, inner, 0)
        return carry

    lax.fori_loop(0, GRPS, body, 0)
    plsc.subcore_barrier()

    pltpu.sync_copy(acc.at[rows], out.at[c, rows])
    if with_hist:
        pltpu.sync_copy(hist, degout.at[w])


def _make_segsum(with_hist):
    out_type = [jax.ShapeDtypeStruct((NC, NPAD, D), jnp.float32)]
    scratch = [
        pltpu.VMEM_SHARED((NPAD, D), jnp.float32),
        pltpu.VMEM((GRP, CHUNK), jnp.int32),
        pltpu.VMEM((GRP, CHUNK), jnp.int32),
        pltpu.VMEM((CHUNK, D), jnp.float32),
    ]
    if with_hist:
        out_type.append(jax.ShapeDtypeStruct((NW, NPAD), jnp.float32))
        scratch.append(pltpu.VMEM((NPAD,), jnp.float32))

    mesh = plsc.VectorSubcoreMesh(core_axis_name="c", subcore_axis_name="s")
    return pl.kernel(
        functools.partial(_segsum_body, with_hist),
        mesh=mesh,
        out_type=out_type,
        scratch_types=scratch,
        compiler_params=pltpu.CompilerParams(needs_layout_passes=False),
    )


def _dense_body(s0, s1, dh, x, wl, wr, bl, o):
    deg = jnp.sum(dh[...], axis=1)
    inv = 1.0 / jnp.maximum(deg, 1.0)
    mean = (s0[...] + s1[...]) * inv[:, None]
    h = lax.dot_general(mean, wl[...], (((1,), (1,)), ((), ())),
                        preferred_element_type=jnp.float32)
    h += lax.dot_general(x[...], wr[...], (((1,), (1,)), ((), ())),
                         preferred_element_type=jnp.float32)
    h += bl[...]
    o[...] = jnp.where(h > 0, h, 0.01 * h)


_BLK = 1000


def _dense(s0, s1, dh, x, wl, wr, bl):
    grid = N // _BLK
    row = pl.BlockSpec((_BLK, D), lambda i: (i, 0))
    deg = pl.BlockSpec((_BLK, NW), lambda i: (i, 0))
    w = pl.BlockSpec((D, D), lambda i: (0, 0))
    b = pl.BlockSpec((1, D), lambda i: (0, 0))
    return pl.pallas_call(
        _dense_body,
        grid=(grid,),
        in_specs=[row, row, deg, row, w, w, b],
        out_specs=row,
        out_shape=jax.ShapeDtypeStruct((N, D), jnp.float32),
    )(s0, s1, dh, x, wl, wr, bl.reshape(1, D))


def kernel(x, edge_index, Wl1, bl1, Wr1, Wl2, bl2, Wr2):
    ei = edge_index.astype(jnp.int32)
    pad = EPAD - E
    src = jnp.concatenate([ei[0], jnp.zeros((pad,), jnp.int32)])
    dst = jnp.concatenate([ei[1], jnp.full((pad,), N, jnp.int32)])
    srcg = src.reshape(NW, CHUNKS, CHUNK)
    dstg = dst.reshape(NW, CHUNKS, CHUNK)

    zeros = jnp.zeros((NPAD, D), jnp.float32)
    zeros1 = jnp.zeros((NPAD,), jnp.float32)

    parts, deghist = _make_segsum(True)(x, srcg, dstg, zeros, zeros1)
    dh = deghist[:, :N].T
    h = _dense(parts[0, :N], parts[1, :N], dh, x, Wl1, Wr1, bl1)

    parts2, = _make_segsum(False)(h, srcg, dstg, zeros)
    return _dense(parts2[0, :N], parts2[1, :N], dh, h, Wl2, Wr2, bl2)
